# Initial kernel scaffold; baseline (speedup 1.0000x reference)
import jax
import jax.numpy as jnp
from jax.experimental import pallas as pl


def kernel(x, tok_emb_weight):
    raise NotImplementedError("write your pallas kernel here")


# speedup vs baseline: 1.7853x; 1.7853x over previous
import functools

import jax
import jax.numpy as jnp
from jax import lax
from jax.experimental import pallas as pl
from jax.experimental.pallas import tpu as pltpu
from jax.experimental.pallas import tpu_sc as plsc

NC = 2
NS = 16
NW = NC * NS
CHUNK = 32
NBUF = 2


def kernel(x, tok_emb_weight):
    B, S = x.shape
    V, D = tok_emb_weight.shape
    n = B * S
    bp = n // NW
    nch = bp // CHUNK
    assert bp * NW == n and nch * CHUNK == bp and nch % NBUF == 0

    idx = x.reshape(NW, nch, CHUNK).astype(jnp.int32)
    mesh = plsc.VectorSubcoreMesh(core_axis_name="c", subcore_axis_name="s")

    @functools.partial(
        pl.kernel,
        mesh=mesh,
        out_type=jax.ShapeDtypeStruct((n, D), jnp.float32),
        scratch_types=[
            pltpu.VMEM((nch, CHUNK), jnp.int32),
            pltpu.VMEM((NBUF, CHUNK, D), jnp.float32),
            pltpu.SemaphoreType.DMA,
            pltpu.SemaphoreType.DMA,
        ],
    )
    def emb(table_hbm, idx_hbm, out_hbm, idx_v, rows_v, gsem, osem):
        wid = lax.axis_index("s") * NC + lax.axis_index("c")
        base = wid * bp
        pltpu.sync_copy(idx_hbm.at[wid], idx_v)

        def gather(ch, b):
            return pltpu.make_async_copy(
                table_hbm.at[idx_v.at[ch]], rows_v.at[b], gsem)

        def put(ch, b):
            return pltpu.make_async_copy(
                rows_v.at[b], out_hbm.at[pl.ds(base + ch * CHUNK, CHUNK)],
                osem)

        for b in range(NBUF):
            gather(b, b).start()

        def body(g, carry):
            for b in range(NBUF):
                ch = g * NBUF + b
                gather(ch, b).wait()
                put(ch, b).start()
                put(ch, b).wait()
                gather(ch + NBUF, b).start()
            return carry

        lax.fori_loop(0, nch // NBUF - 1, body, 0)

        for b in range(NBUF):
            ch = nch - NBUF + b
            gather(ch, b).wait()
            put(ch, b).start()
            put(ch, b).wait()

    out = emb(tok_emb_weight, idx)
    return out.reshape(B, S, D)


# speedup vs baseline: 1.7957x; 1.0058x over previous
import functools

import jax
import jax.numpy as jnp
from jax import lax
from jax.experimental import pallas as pl
from jax.experimental.pallas import tpu as pltpu
from jax.experimental.pallas import tpu_sc as plsc

NC = 2
NS = 16
NW = NC * NS
CHUNK = 32
NBUF = 3


def kernel(x, tok_emb_weight):
    B, S = x.shape
    V, D = tok_emb_weight.shape
    n = B * S
    bp = n // NW
    nch = bp // CHUNK
    assert bp * NW == n and nch * CHUNK == bp and nch >= 4

    idx = x.reshape(NW, nch, CHUNK).astype(jnp.int32)
    mesh = plsc.VectorSubcoreMesh(core_axis_name="c", subcore_axis_name="s")

    @functools.partial(
        pl.kernel,
        mesh=mesh,
        out_type=jax.ShapeDtypeStruct((n, D), jnp.float32),
        scratch_types=[
            pltpu.VMEM((nch, CHUNK), jnp.int32),
            pltpu.VMEM((NBUF, CHUNK, D), jnp.float32),
            pltpu.SemaphoreType.DMA,
            pltpu.SemaphoreType.DMA,
        ],
    )
    def emb(table_hbm, idx_hbm, out_hbm, idx_v, rows_v, gsem, osem):
        wid = lax.axis_index("s") * NC + lax.axis_index("c")
        base = wid * bp
        pltpu.sync_copy(idx_hbm.at[wid], idx_v)

        def gather(ch):
            return pltpu.make_async_copy(
                table_hbm.at[idx_v.at[ch]], rows_v.at[ch % NBUF], gsem)

        def put(ch):
            return pltpu.make_async_copy(
                rows_v.at[ch % NBUF],
                out_hbm.at[pl.ds(base + ch * CHUNK, CHUNK)], osem)

        gather(0).start()
        gather(1).start()

        gather(0).wait()
        put(0).start()
        gather(2).start()

        def body(ch, carry):
            gather(ch).wait()
            put(ch).start()
            put(ch - 1).wait()
            gather(ch + 2).start()
            return carry

        lax.fori_loop(1, nch - 2, body, 0)

        for ch in (nch - 2, nch - 1):
            gather(ch).wait()
            put(ch).start()
            put(ch - 1).wait()
        put(nch - 1).wait()

    out = emb(tok_emb_weight, idx)
    return out.reshape(B, S, D)
